# baseline (speedup 1.0000x reference)
import functools

import jax
import jax.numpy as jnp
from jax.experimental import pallas as pl
from jax.experimental.pallas import tpu as pltpu

_LANES = 128
_SUBLANES = 8
_NUM_CORES = 2
_TILE_ROWS = 4096


def _rmse_acc_kernel(yhat_ref, y_ref, acc_ref):
    i = pl.program_id(1)
    d = yhat_ref[...] - y_ref[...]
    sq = d * d
    partial = jnp.sum(sq.reshape(-1, _SUBLANES, _LANES), axis=0)

    @pl.when(i == 0)
    def _():
        acc_ref[...] = partial

    @pl.when(i != 0)
    def _():
        acc_ref[...] += partial


@functools.partial(jax.jit, static_argnames=("eps",))
def _rmse(yhat, y, eps=1e-6):
    n_elems = yhat.size
    flat_yhat = yhat.reshape(-1)
    flat_y = y.reshape(-1)

    chunk = _NUM_CORES * _TILE_ROWS * _LANES
    n_pad = (-n_elems) % chunk
    if n_pad:
        flat_yhat = jnp.pad(flat_yhat, (0, n_pad))
        flat_y = jnp.pad(flat_y, (0, n_pad))

    n_rows = (n_elems + n_pad) // _LANES
    steps = n_rows // (_NUM_CORES * _TILE_ROWS)

    yhat2d = flat_yhat.reshape(n_rows, _LANES).astype(jnp.float32)
    y2d = flat_y.reshape(n_rows, _LANES).astype(jnp.float32)

    in_map = lambda c, i: (c * steps + i, 0)
    partials = pl.pallas_call(
        _rmse_acc_kernel,
        out_shape=jax.ShapeDtypeStruct((_NUM_CORES, _SUBLANES, _LANES),
                                       jnp.float32),
        grid=(_NUM_CORES, steps),
        in_specs=[
            pl.BlockSpec((_TILE_ROWS, _LANES), in_map),
            pl.BlockSpec((_TILE_ROWS, _LANES), in_map),
        ],
        out_specs=pl.BlockSpec((None, _SUBLANES, _LANES),
                               lambda c, i: (c, 0, 0)),
        compiler_params=pltpu.CompilerParams(
            dimension_semantics=("parallel", "arbitrary")),
    )(yhat2d, y2d)

    mse = jnp.sum(partials) / jnp.float32(n_elems)
    return jnp.sqrt(mse + jnp.float32(eps))


def kernel(yhat, y):
    return _rmse(yhat, y)


# speedup vs baseline: 1.0260x; 1.0260x over previous
import functools

import jax
import jax.numpy as jnp
from jax.experimental import pallas as pl
from jax.experimental.pallas import tpu as pltpu

_LANES = 128
_SUBLANES = 8
_NUM_CORES = 2
_NUM_STRIPES = 4
_TILE_ROWS = 1024


def _rmse_acc_kernel(*refs):
    in_refs = refs[:-1]
    acc_ref = refs[-1]
    i = pl.program_id(1)

    partial = jnp.zeros((_SUBLANES, _LANES), jnp.float32)
    for k in range(_NUM_STRIPES):
        d = in_refs[2 * k][...] - in_refs[2 * k + 1][...]
        sq = d * d
        partial = partial + jnp.sum(sq.reshape(-1, _SUBLANES, _LANES), axis=0)

    @pl.when(i == 0)
    def _():
        acc_ref[...] = partial

    @pl.when(i != 0)
    def _():
        acc_ref[...] += partial


@functools.partial(jax.jit, static_argnames=("eps",))
def _rmse(yhat, y, eps=1e-6):
    n_elems = yhat.size
    flat_yhat = yhat.reshape(-1)
    flat_y = y.reshape(-1)

    chunk = _NUM_CORES * _NUM_STRIPES * _TILE_ROWS * _LANES
    n_pad = (-n_elems) % chunk
    if n_pad:
        flat_yhat = jnp.pad(flat_yhat, (0, n_pad))
        flat_y = jnp.pad(flat_y, (0, n_pad))

    n_rows = (n_elems + n_pad) // _LANES
    steps = n_rows // (_NUM_CORES * _NUM_STRIPES * _TILE_ROWS)
    stripe_blocks = n_rows // (_NUM_STRIPES * _TILE_ROWS)
    blocks_per_core = steps

    yhat2d = flat_yhat.reshape(n_rows, _LANES).astype(jnp.float32)
    y2d = flat_y.reshape(n_rows, _LANES).astype(jnp.float32)

    def make_map(k):
        base = k * stripe_blocks
        return lambda c, i: (base + c * blocks_per_core + i, 0)

    in_specs = []
    operands = []
    for k in range(_NUM_STRIPES):
        m = make_map(k)
        in_specs.append(pl.BlockSpec((_TILE_ROWS, _LANES), m))
        in_specs.append(pl.BlockSpec((_TILE_ROWS, _LANES), m))
        operands.append(yhat2d)
        operands.append(y2d)

    partials = pl.pallas_call(
        _rmse_acc_kernel,
        out_shape=jax.ShapeDtypeStruct((_NUM_CORES, _SUBLANES, _LANES),
                                       jnp.float32),
        grid=(_NUM_CORES, steps),
        in_specs=in_specs,
        out_specs=pl.BlockSpec((None, _SUBLANES, _LANES),
                               lambda c, i: (c, 0, 0)),
        compiler_params=pltpu.CompilerParams(
            dimension_semantics=("parallel", "arbitrary")),
    )(*operands)

    mse = jnp.sum(partials) / jnp.float32(n_elems)
    return jnp.sqrt(mse + jnp.float32(eps))


def kernel(yhat, y):
    return _rmse(yhat, y)


# speedup vs baseline: 1.0759x; 1.0486x over previous
import functools

import jax
import jax.numpy as jnp
from jax.experimental import pallas as pl
from jax.experimental.pallas import tpu as pltpu

_LANES = 128
_SUBLANES = 8
_NUM_STRIPES = 4
_TILE_ROWS = 1024


def _rmse_kernel(*refs, steps, inv_n, eps):
    in_refs = refs[:-2]
    out_ref = refs[-2]
    acc_ref = refs[-1]
    i = pl.program_id(0)

    partial = jnp.zeros((_SUBLANES, _LANES), jnp.float32)
    for k in range(_NUM_STRIPES):
        d = in_refs[2 * k][...] - in_refs[2 * k + 1][...]
        sq = d * d
        partial = partial + jnp.sum(sq.reshape(-1, _SUBLANES, _LANES), axis=0)

    @pl.when(i == 0)
    def _():
        acc_ref[...] = partial

    @pl.when(i != 0)
    def _():
        acc_ref[...] += partial

    @pl.when(i == steps - 1)
    def _():
        total = jnp.sum(acc_ref[...])
        out_ref[0, 0] = jnp.sqrt(total * jnp.float32(inv_n) + jnp.float32(eps))


@functools.partial(jax.jit, static_argnames=("eps",))
def _rmse(yhat, y, eps=1e-6):
    n_elems = yhat.size
    flat_yhat = yhat.reshape(-1)
    flat_y = y.reshape(-1)

    chunk = _NUM_STRIPES * _TILE_ROWS * _LANES
    n_pad = (-n_elems) % chunk
    if n_pad:
        flat_yhat = jnp.pad(flat_yhat, (0, n_pad))
        flat_y = jnp.pad(flat_y, (0, n_pad))

    n_rows = (n_elems + n_pad) // _LANES
    steps = n_rows // (_NUM_STRIPES * _TILE_ROWS)
    stripe_blocks = steps

    yhat2d = flat_yhat.reshape(n_rows, _LANES).astype(jnp.float32)
    y2d = flat_y.reshape(n_rows, _LANES).astype(jnp.float32)

    def make_map(k):
        base = k * stripe_blocks
        return lambda i: (base + i, 0)

    in_specs = []
    operands = []
    for k in range(_NUM_STRIPES):
        m = make_map(k)
        in_specs.append(pl.BlockSpec((_TILE_ROWS, _LANES), m))
        in_specs.append(pl.BlockSpec((_TILE_ROWS, _LANES), m))
        operands.append(yhat2d)
        operands.append(y2d)

    body = functools.partial(
        _rmse_kernel, steps=steps, inv_n=1.0 / n_elems, eps=float(eps))

    out = pl.pallas_call(
        body,
        out_shape=jax.ShapeDtypeStruct((1, 1), jnp.float32),
        grid=(steps,),
        in_specs=in_specs,
        out_specs=pl.BlockSpec(memory_space=pltpu.SMEM),
        scratch_shapes=[pltpu.VMEM((_SUBLANES, _LANES), jnp.float32)],
        compiler_params=pltpu.CompilerParams(
            dimension_semantics=("arbitrary",)),
    )(*operands)

    return out.reshape(())


def kernel(yhat, y):
    return _rmse(yhat, y)


# speedup vs baseline: 1.0829x; 1.0065x over previous
import functools

import jax
import jax.numpy as jnp
from jax.experimental import pallas as pl
from jax.experimental.pallas import tpu as pltpu

_LANES = 128
_SUBLANES = 8
_CHUNK_ROWS = 4096
_NUM_BUFS = 4


def _rmse_kernel(yhat_hbm, y_hbm, out_ref, ybuf, tbuf, ysem, tsem,
                 *, n_chunks, inv_n, eps):
    def copy_in(c):
        s = c % _NUM_BUFS
        r0 = c * _CHUNK_ROWS
        return (
            pltpu.make_async_copy(
                yhat_hbm.at[pl.ds(r0, _CHUNK_ROWS), :], ybuf.at[s],
                ysem.at[s]),
            pltpu.make_async_copy(
                y_hbm.at[pl.ds(r0, _CHUNK_ROWS), :], tbuf.at[s],
                tsem.at[s]),
        )

    for c in range(min(_NUM_BUFS, n_chunks)):
        a, b = copy_in(c)
        a.start()
        b.start()

    acc = jnp.zeros((_SUBLANES, _LANES), jnp.float32)
    for c in range(n_chunks):
        s = c % _NUM_BUFS
        a, b = copy_in(c)
        a.wait()
        b.wait()
        d = ybuf[s] - tbuf[s]
        sq = d * d
        acc = acc + jnp.sum(sq.reshape(-1, _SUBLANES, _LANES), axis=0)
        if c + _NUM_BUFS < n_chunks:
            a, b = copy_in(c + _NUM_BUFS)
            a.start()
            b.start()

    total = jnp.sum(acc)
    out_ref[0, 0] = jnp.sqrt(total * jnp.float32(inv_n) + jnp.float32(eps))


@functools.partial(jax.jit, static_argnames=("eps",))
def _rmse(yhat, y, eps=1e-6):
    n_elems = yhat.size
    flat_yhat = yhat.reshape(-1)
    flat_y = y.reshape(-1)

    chunk = _CHUNK_ROWS * _LANES
    n_pad = (-n_elems) % chunk
    if n_pad:
        flat_yhat = jnp.pad(flat_yhat, (0, n_pad))
        flat_y = jnp.pad(flat_y, (0, n_pad))

    n_rows = (n_elems + n_pad) // _LANES
    n_chunks = n_rows // _CHUNK_ROWS

    yhat2d = flat_yhat.reshape(n_rows, _LANES).astype(jnp.float32)
    y2d = flat_y.reshape(n_rows, _LANES).astype(jnp.float32)

    body = functools.partial(
        _rmse_kernel, n_chunks=n_chunks, inv_n=1.0 / n_elems, eps=float(eps))

    out = pl.pallas_call(
        body,
        out_shape=jax.ShapeDtypeStruct((1, 1), jnp.float32),
        in_specs=[
            pl.BlockSpec(memory_space=pl.ANY),
            pl.BlockSpec(memory_space=pl.ANY),
        ],
        out_specs=pl.BlockSpec(memory_space=pltpu.SMEM),
        scratch_shapes=[
            pltpu.VMEM((_NUM_BUFS, _CHUNK_ROWS, _LANES), jnp.float32),
            pltpu.VMEM((_NUM_BUFS, _CHUNK_ROWS, _LANES), jnp.float32),
            pltpu.SemaphoreType.DMA((_NUM_BUFS,)),
            pltpu.SemaphoreType.DMA((_NUM_BUFS,)),
        ],
    )(yhat2d, y2d)

    return out.reshape(())


def kernel(yhat, y):
    return _rmse(yhat, y)


# speedup vs baseline: 1.0840x; 1.0009x over previous
import functools

import jax
import jax.numpy as jnp
from jax.experimental import pallas as pl
from jax.experimental.pallas import tpu as pltpu

_LANES = 128
_SUBLANES = 8
_CHUNK_ROWS = 4096
_NUM_BUFS = 4


def _rmse_kernel(yhat_hbm, y_hbm, out_ref, ybuf, tbuf, ysem, tsem,
                 *, n_chunks, inv_n, eps):
    def copy_in(c):
        s = c % _NUM_BUFS
        r0 = c * _CHUNK_ROWS
        return (
            pltpu.make_async_copy(
                yhat_hbm.at[pl.ds(r0, _CHUNK_ROWS), :], ybuf.at[s],
                ysem.at[s]),
            pltpu.make_async_copy(
                y_hbm.at[pl.ds(r0, _CHUNK_ROWS), :], tbuf.at[s],
                tsem.at[s]),
        )

    for c in range(min(_NUM_BUFS, n_chunks)):
        a, b = copy_in(c)
        a.start()
        b.start(priority=1)

    acc = jnp.zeros((_SUBLANES, _LANES), jnp.float32)
    for c in range(n_chunks):
        s = c % _NUM_BUFS
        a, b = copy_in(c)
        a.wait()
        b.wait()
        d = ybuf[s] - tbuf[s]
        sq = d * d
        acc = acc + jnp.sum(sq.reshape(-1, _SUBLANES, _LANES), axis=0)
        if c + _NUM_BUFS < n_chunks:
            a, b = copy_in(c + _NUM_BUFS)
            a.start()
            b.start(priority=1)

    total = jnp.sum(acc)
    out_ref[0, 0] = jnp.sqrt(total * jnp.float32(inv_n) + jnp.float32(eps))


@functools.partial(jax.jit, static_argnames=("eps",))
def _rmse(yhat, y, eps=1e-6):
    n_elems = yhat.size
    flat_yhat = yhat.reshape(-1)
    flat_y = y.reshape(-1)

    chunk = _CHUNK_ROWS * _LANES
    n_pad = (-n_elems) % chunk
    if n_pad:
        flat_yhat = jnp.pad(flat_yhat, (0, n_pad))
        flat_y = jnp.pad(flat_y, (0, n_pad))

    n_rows = (n_elems + n_pad) // _LANES
    n_chunks = n_rows // _CHUNK_ROWS

    yhat2d = flat_yhat.reshape(n_rows, _LANES).astype(jnp.float32)
    y2d = flat_y.reshape(n_rows, _LANES).astype(jnp.float32)

    body = functools.partial(
        _rmse_kernel, n_chunks=n_chunks, inv_n=1.0 / n_elems, eps=float(eps))

    out = pl.pallas_call(
        body,
        out_shape=jax.ShapeDtypeStruct((1, 1), jnp.float32),
        in_specs=[
            pl.BlockSpec(memory_space=pl.ANY),
            pl.BlockSpec(memory_space=pl.ANY),
        ],
        out_specs=pl.BlockSpec(memory_space=pltpu.SMEM),
        scratch_shapes=[
            pltpu.VMEM((_NUM_BUFS, _CHUNK_ROWS, _LANES), jnp.float32),
            pltpu.VMEM((_NUM_BUFS, _CHUNK_ROWS, _LANES), jnp.float32),
            pltpu.SemaphoreType.DMA((_NUM_BUFS,)),
            pltpu.SemaphoreType.DMA((_NUM_BUFS,)),
        ],
    )(yhat2d, y2d)

    return out.reshape(())


def kernel(yhat, y):
    return _rmse(yhat, y)
